# baseline (speedup 1.0000x reference)
import functools

import jax
import jax.numpy as jnp
from jax import lax
from jax.experimental import pallas as pl
from jax.experimental.pallas import tpu as pltpu
from jax.experimental.pallas import tpu_sc as plsc

ENTITY_NUM = 100000
DIM = 64
B = 1024
N_TILE = 1024

_info = plsc.get_sparse_core_info()
_NC, _NS = _info.num_cores, _info.num_subcores
_NW = _NC * _NS
_BPW = B // _NW


def _make_sc_gather():
    mesh = plsc.VectorSubcoreMesh(core_axis_name="c", subcore_axis_name="s")

    @functools.partial(
        pl.kernel,
        mesh=mesh,
        out_type=[
            jax.ShapeDtypeStruct((B, DIM), jnp.float32),
            jax.ShapeDtypeStruct((B, DIM), jnp.float32),
            jax.ShapeDtypeStruct((B,), jnp.float32),
        ],
        scratch_types=[
            pltpu.VMEM((_BPW,), jnp.int32),
            pltpu.VMEM((_BPW,), jnp.int32),
            pltpu.VMEM((_BPW, DIM), jnp.float32),
            pltpu.VMEM((_BPW, DIM), jnp.float32),
            pltpu.VMEM((_BPW,), jnp.float32),
            pltpu.SemaphoreType.DMA,
            pltpu.SemaphoreType.DMA,
            pltpu.SemaphoreType.DMA,
        ],
        compiler_params=pltpu.CompilerParams(use_tc_tiling_on_sc=False),
    )
    def sc_gather(src_hbm, rel_hbm, ent_hbm, rele_hbm, hb_hbm,
                  out_h, out_r, out_hb,
                  src_v, rel_v, h_rows, r_rows, hb_rows,
                  sem_h, sem_r, sem_hb):
        wid = lax.axis_index("s") * _NC + lax.axis_index("c")
        base = wid * _BPW
        pltpu.sync_copy(src_hbm.at[pl.ds(base, _BPW)], src_v)
        pltpu.sync_copy(rel_hbm.at[pl.ds(base, _BPW)], rel_v)
        cp_h = pltpu.async_copy(ent_hbm.at[src_v], h_rows, sem_h)
        cp_r = pltpu.async_copy(rele_hbm.at[rel_v], r_rows, sem_r)
        cp_b = pltpu.async_copy(hb_hbm.at[src_v], hb_rows, sem_hb)
        cp_h.wait()
        cp_r.wait()
        cp_b.wait()
        pltpu.sync_copy(h_rows, out_h.at[pl.ds(base, _BPW)])
        pltpu.sync_copy(r_rows, out_r.at[pl.ds(base, _BPW)])
        pltpu.sync_copy(hb_rows, out_hb.at[pl.ds(base, _BPW)])

    return sc_gather


def _tc_score(h_ref, r_ref, hb_ref, tail_ref, ent_ref, out_ref):
    hr = h_ref[...] * r_ref[...]
    acc = lax.dot_general(hr, ent_ref[...], (((1,), (1,)), ((), ())),
                          preferred_element_type=jnp.float32)
    out_ref[...] = acc + hb_ref[...] + tail_ref[...]


def kernel(src, rel, ent_embed, rel_embed, head_bias, tail_bias):
    n = ent_embed.shape[0]
    src_f = src.reshape(B).astype(jnp.int32)
    rel_f = rel.reshape(B).astype(jnp.int32)

    sc_gather = _make_sc_gather()
    h_rows, r_rows, hb = sc_gather(src_f, rel_f, ent_embed, rel_embed,
                                   head_bias.reshape(n))
    hb = hb.reshape(B, 1)

    tail_row = tail_bias.reshape(1, n)
    nb = pl.cdiv(n, N_TILE)
    out = pl.pallas_call(
        _tc_score,
        grid=(nb,),
        in_specs=[
            pl.BlockSpec((B, DIM), lambda j: (0, 0)),
            pl.BlockSpec((B, DIM), lambda j: (0, 0)),
            pl.BlockSpec((B, 1), lambda j: (0, 0)),
            pl.BlockSpec((1, N_TILE), lambda j: (0, j)),
            pl.BlockSpec((N_TILE, DIM), lambda j: (j, 0)),
        ],
        out_specs=pl.BlockSpec((B, N_TILE), lambda j: (0, j)),
        out_shape=jax.ShapeDtypeStruct((B, n), jnp.float32),
        compiler_params=pltpu.CompilerParams(
            dimension_semantics=("arbitrary",)),
    )(h_rows, r_rows, hb, tail_row, ent_embed)
    return out


# speedup vs baseline: 1.0354x; 1.0354x over previous
import functools

import jax
import jax.numpy as jnp
from jax import lax
from jax.experimental import pallas as pl
from jax.experimental.pallas import tpu as pltpu
from jax.experimental.pallas import tpu_sc as plsc

ENTITY_NUM = 100000
DIM = 64
B = 1024
N_TILE = 2048

_info = plsc.get_sparse_core_info()
_NC, _NS = _info.num_cores, _info.num_subcores
_NW = _NC * _NS
_BPW = B // _NW


def _make_sc_gather():
    mesh = plsc.VectorSubcoreMesh(core_axis_name="c", subcore_axis_name="s")

    @functools.partial(
        pl.kernel,
        mesh=mesh,
        out_type=[
            jax.ShapeDtypeStruct((B, DIM), jnp.float32),
            jax.ShapeDtypeStruct((B, DIM), jnp.float32),
            jax.ShapeDtypeStruct((B,), jnp.float32),
        ],
        scratch_types=[
            pltpu.VMEM((_BPW,), jnp.int32),
            pltpu.VMEM((_BPW,), jnp.int32),
            pltpu.VMEM((_BPW, DIM), jnp.float32),
            pltpu.VMEM((_BPW, DIM), jnp.float32),
            pltpu.VMEM((_BPW,), jnp.float32),
            pltpu.SemaphoreType.DMA,
            pltpu.SemaphoreType.DMA,
            pltpu.SemaphoreType.DMA,
        ],
        compiler_params=pltpu.CompilerParams(use_tc_tiling_on_sc=False),
    )
    def sc_gather(src_hbm, rel_hbm, ent_hbm, rele_hbm, hb_hbm,
                  out_h, out_r, out_hb,
                  src_v, rel_v, h_rows, r_rows, hb_rows,
                  sem_h, sem_r, sem_hb):
        wid = lax.axis_index("s") * _NC + lax.axis_index("c")
        base = wid * _BPW
        pltpu.sync_copy(src_hbm.at[pl.ds(base, _BPW)], src_v)
        pltpu.sync_copy(rel_hbm.at[pl.ds(base, _BPW)], rel_v)
        cp_h = pltpu.async_copy(ent_hbm.at[src_v], h_rows, sem_h)
        cp_r = pltpu.async_copy(rele_hbm.at[rel_v], r_rows, sem_r)
        cp_b = pltpu.async_copy(hb_hbm.at[src_v], hb_rows, sem_hb)
        cp_h.wait()
        cp_r.wait()
        cp_b.wait()
        pltpu.sync_copy(h_rows, out_h.at[pl.ds(base, _BPW)])
        pltpu.sync_copy(r_rows, out_r.at[pl.ds(base, _BPW)])
        pltpu.sync_copy(hb_rows, out_hb.at[pl.ds(base, _BPW)])

    return sc_gather


def _tc_score(h_ref, r_ref, hb_ref, tail_ref, ent_ref, out_ref):
    hr = h_ref[...] * r_ref[...]
    acc = lax.dot_general(hr, ent_ref[...], (((1,), (1,)), ((), ())),
                          preferred_element_type=jnp.float32)
    out_ref[...] = acc + hb_ref[...] + tail_ref[...]


def kernel(src, rel, ent_embed, rel_embed, head_bias, tail_bias):
    n = ent_embed.shape[0]
    src_f = src.reshape(B).astype(jnp.int32)
    rel_f = rel.reshape(B).astype(jnp.int32)

    sc_gather = _make_sc_gather()
    h_rows, r_rows, hb = sc_gather(src_f, rel_f, ent_embed, rel_embed,
                                   head_bias.reshape(n))
    hb = hb.reshape(B, 1)

    tail_row = tail_bias.reshape(1, n)
    nb = pl.cdiv(n, N_TILE)
    out = pl.pallas_call(
        _tc_score,
        grid=(nb,),
        in_specs=[
            pl.BlockSpec((B, DIM), lambda j: (0, 0)),
            pl.BlockSpec((B, DIM), lambda j: (0, 0)),
            pl.BlockSpec((B, 1), lambda j: (0, 0)),
            pl.BlockSpec((1, N_TILE), lambda j: (0, j)),
            pl.BlockSpec((N_TILE, DIM), lambda j: (j, 0)),
        ],
        out_specs=pl.BlockSpec((B, N_TILE), lambda j: (0, j)),
        out_shape=jax.ShapeDtypeStruct((B, n), jnp.float32),
        compiler_params=pltpu.CompilerParams(
            dimension_semantics=("parallel",)),
    )(h_rows, r_rows, hb, tail_row, ent_embed)
    return out


# speedup vs baseline: 3.0837x; 2.9783x over previous
import functools

import jax
import jax.numpy as jnp
from jax import lax
from jax.experimental import pallas as pl
from jax.experimental.pallas import tpu as pltpu
from jax.experimental.pallas import tpu_sc as plsc

DIM = 64
B = 1024
N_TILE = 2048

_info = plsc.get_sparse_core_info()
_NC, _NS = _info.num_cores, _info.num_subcores
_NW = _NC * _NS
_DPW = DIM // _NW
_BPW = B // _NW
_NCHUNK = B // 128


def _make_sc_gather(ent_n, rel_n):
    mesh = plsc.VectorSubcoreMesh(core_axis_name="c", subcore_axis_name="s")

    @functools.partial(
        pl.kernel,
        mesh=mesh,
        out_type=[
            jax.ShapeDtypeStruct((DIM, B), jnp.float32),
            jax.ShapeDtypeStruct((B,), jnp.float32),
        ],
        scratch_types=[
            pltpu.VMEM((B,), jnp.int32),
            pltpu.VMEM((B,), jnp.int32),
            pltpu.VMEM((_DPW, B), jnp.int32),
            pltpu.VMEM((_DPW, B), jnp.int32),
            pltpu.VMEM((_DPW, B), jnp.float32),
            pltpu.VMEM((_DPW, B), jnp.float32),
            pltpu.VMEM((_BPW,), jnp.float32),
            pltpu.SemaphoreType.DMA,
            pltpu.SemaphoreType.DMA,
            pltpu.SemaphoreType.DMA,
        ],
        compiler_params=pltpu.CompilerParams(use_tc_tiling_on_sc=False),
    )
    def sc_gather(src_hbm, rel_hbm, entt_hbm, relt_hbm, hb_hbm,
                  out_hrt, out_hb,
                  src_v, rel_v, idx_e, idx_r, h_v, r_v, hb_v,
                  sem_e, sem_r, sem_b):
        wid = lax.axis_index("s") * _NC + lax.axis_index("c")
        base_d = wid * _DPW
        pltpu.sync_copy(src_hbm, src_v)
        pltpu.sync_copy(rel_hbm, rel_v)

        def build(c, _):
            sl = pl.ds(c * 16, 16)
            s16 = src_v[sl]
            r16 = rel_v[sl]
            for di in range(_DPW):
                idx_e[di, sl] = s16 + (base_d + di) * ent_n
                idx_r[di, sl] = r16 + (base_d + di) * rel_n
            return ()

        lax.fori_loop(0, B // 16, build, ())

        cp_b = pltpu.async_copy(
            hb_hbm.at[src_v.at[pl.ds(wid * _BPW, _BPW)]], hb_v, sem_b)
        copies = []
        for di in range(_DPW):
            for c in range(_NCHUNK):
                sl = pl.ds(c * 128, 128)
                copies.append(pltpu.async_copy(
                    entt_hbm.at[idx_e.at[di, sl]], h_v.at[di, sl], sem_e))
                copies.append(pltpu.async_copy(
                    relt_hbm.at[idx_r.at[di, sl]], r_v.at[di, sl], sem_r))
        for cp in copies:
            cp.wait()

        def mul(c, _):
            sl = pl.ds(c * 16, 16)
            for di in range(_DPW):
                h_v[di, sl] = h_v[di, sl] * r_v[di, sl]
            return ()

        lax.fori_loop(0, B // 16, mul, ())

        pltpu.sync_copy(h_v, out_hrt.at[pl.ds(base_d, _DPW)])
        cp_b.wait()
        pltpu.sync_copy(hb_v, out_hb.at[pl.ds(wid * _BPW, _BPW)])

    return sc_gather


def _tc_score(entt_ref, hrt_ref, hb_ref, tail_ref, out_ref):
    acc = lax.dot_general(entt_ref[...], hrt_ref[...], (((0,), (0,)), ((), ())),
                          preferred_element_type=jnp.float32)
    ones_row = jnp.full((1, B), 1.0, dtype=jnp.float32)
    tcol = lax.dot_general(tail_ref[...], ones_row, (((0,), (0,)), ((), ())),
                           preferred_element_type=jnp.float32)
    out_ref[...] = acc + tcol + hb_ref[...]


def kernel(src, rel, ent_embed, rel_embed, head_bias, tail_bias):
    n = ent_embed.shape[0]
    rn = rel_embed.shape[0]
    src_f = src.reshape(B).astype(jnp.int32)
    rel_f = rel.reshape(B).astype(jnp.int32)
    entt = ent_embed.T
    relt = rel_embed.T

    sc_gather = _make_sc_gather(n, rn)
    hrt, hb = sc_gather(src_f, rel_f, entt.reshape(-1), relt.reshape(-1),
                        head_bias.reshape(n))

    nb = pl.cdiv(n, N_TILE)
    out_t = pl.pallas_call(
        _tc_score,
        grid=(nb,),
        in_specs=[
            pl.BlockSpec((DIM, N_TILE), lambda j: (0, j)),
            pl.BlockSpec((DIM, B), lambda j: (0, 0)),
            pl.BlockSpec((1, B), lambda j: (0, 0)),
            pl.BlockSpec((1, N_TILE), lambda j: (0, j)),
        ],
        out_specs=pl.BlockSpec((N_TILE, B), lambda j: (j, 0)),
        out_shape=jax.ShapeDtypeStruct((n, B), jnp.float32),
        compiler_params=pltpu.CompilerParams(
            dimension_semantics=("parallel",)),
    )(entt, hrt, hb.reshape(1, B), tail_bias.reshape(1, n))
    return out_t.T


# speedup vs baseline: 3.6948x; 1.1982x over previous
import functools

import jax
import jax.numpy as jnp
from jax import lax
from jax.experimental import pallas as pl
from jax.experimental.pallas import tpu as pltpu
from jax.experimental.pallas import tpu_sc as plsc

DIM = 64
B = 1024
N_TILE = 2048

_info = plsc.get_sparse_core_info()
_NC, _NS = _info.num_cores, _info.num_subcores
_NW = _NC * _NS
_DPW = DIM // _NW
_BPW = B // _NW
_NCHUNK = B // 128


def _make_sc_gather(ent_n, rel_n):
    mesh = plsc.VectorSubcoreMesh(core_axis_name="c", subcore_axis_name="s")

    @functools.partial(
        pl.kernel,
        mesh=mesh,
        out_type=[
            jax.ShapeDtypeStruct((DIM, B), jnp.float32),
            jax.ShapeDtypeStruct((B,), jnp.float32),
        ],
        scratch_types=[
            pltpu.VMEM((B,), jnp.int32),
            pltpu.VMEM((B,), jnp.int32),
            pltpu.VMEM((ent_n,), jnp.float32),
            pltpu.VMEM((rel_n,), jnp.float32),
            pltpu.VMEM((_DPW, B), jnp.float32),
            pltpu.VMEM((_BPW,), jnp.float32),
            pltpu.SemaphoreType.DMA,
            pltpu.SemaphoreType.DMA,
            pltpu.SemaphoreType.DMA,
        ],
        compiler_params=pltpu.CompilerParams(use_tc_tiling_on_sc=True,
                                             needs_layout_passes=False),
    )
    def sc_gather(src_hbm, rel_hbm, entt_hbm, relt_hbm, hb_hbm,
                  out_hrt, out_hb,
                  src_v, rel_v, row_v, rrow_v, h_v, hb_v,
                  sem_e, sem_r, sem_b):
        wid = lax.axis_index("s") * _NC + lax.axis_index("c")
        base_d = wid * _DPW
        pltpu.sync_copy(src_hbm, src_v)
        pltpu.sync_copy(rel_hbm, rel_v)
        cp_b = pltpu.async_copy(
            hb_hbm.at[src_v.at[pl.ds(wid * _BPW, _BPW)]], hb_v, sem_b)

        for di in range(_DPW):
            d = base_d + di
            cp_e = pltpu.async_copy(entt_hbm.at[d], row_v, sem_e)
            cp_r = pltpu.async_copy(relt_hbm.at[d], rrow_v, sem_r)
            cp_e.wait()
            cp_r.wait()

            def gather(c, _):
                sl = pl.ds(c * 16, 16)
                h = plsc.load_gather(row_v, [src_v[sl]])
                r = plsc.load_gather(rrow_v, [rel_v[sl]])
                h_v[di, sl] = h * r
                return ()

            lax.fori_loop(0, B // 16, gather, ())

        pltpu.sync_copy(h_v, out_hrt.at[pl.ds(base_d, _DPW)])
        cp_b.wait()
        pltpu.sync_copy(hb_v, out_hb.at[pl.ds(wid * _BPW, _BPW)])

    return sc_gather


def _tc_score(entt_ref, hrt_ref, hb_ref, tail_ref, out_ref):
    acc = lax.dot_general(entt_ref[...], hrt_ref[...], (((0,), (0,)), ((), ())),
                          preferred_element_type=jnp.float32)
    ones_row = jnp.full((1, B), 1.0, dtype=jnp.float32)
    tcol = lax.dot_general(tail_ref[...], ones_row, (((0,), (0,)), ((), ())),
                           preferred_element_type=jnp.float32)
    out_ref[...] = acc + tcol + hb_ref[...]


def kernel(src, rel, ent_embed, rel_embed, head_bias, tail_bias):
    n = ent_embed.shape[0]
    rn = rel_embed.shape[0]
    src_f = src.reshape(B).astype(jnp.int32)
    rel_f = rel.reshape(B).astype(jnp.int32)
    entt = ent_embed.T
    relt = rel_embed.T

    sc_gather = _make_sc_gather(n, rn)
    hrt, hb = sc_gather(src_f, rel_f, entt, relt, head_bias.reshape(n))

    nb = pl.cdiv(n, N_TILE)
    out_t = pl.pallas_call(
        _tc_score,
        grid=(nb,),
        in_specs=[
            pl.BlockSpec((DIM, N_TILE), lambda j: (0, j)),
            pl.BlockSpec((DIM, B), lambda j: (0, 0)),
            pl.BlockSpec((1, B), lambda j: (0, 0)),
            pl.BlockSpec((1, N_TILE), lambda j: (0, j)),
        ],
        out_specs=pl.BlockSpec((N_TILE, B), lambda j: (j, 0)),
        out_shape=jax.ShapeDtypeStruct((n, B), jnp.float32),
        compiler_params=pltpu.CompilerParams(
            dimension_semantics=("parallel",)),
    )(entt, hrt, hb.reshape(1, B), tail_bias.reshape(1, n))
    return out_t.T


# speedup vs baseline: 3.7643x; 1.0188x over previous
import functools

import jax
import jax.numpy as jnp
from jax import lax
from jax.experimental import pallas as pl
from jax.experimental.pallas import tpu as pltpu
from jax.experimental.pallas import tpu_sc as plsc

DIM = 64
B = 1024
N_TILE = 4096

_info = plsc.get_sparse_core_info()
_NC, _NS = _info.num_cores, _info.num_subcores
_NW = _NC * _NS
_DPW = DIM // _NW
_BPW = B // _NW
_NCHUNK = B // 128


def _make_sc_gather(ent_n, rel_n):
    mesh = plsc.VectorSubcoreMesh(core_axis_name="c", subcore_axis_name="s")

    @functools.partial(
        pl.kernel,
        mesh=mesh,
        out_type=[
            jax.ShapeDtypeStruct((DIM, B), jnp.float32),
            jax.ShapeDtypeStruct((B,), jnp.float32),
        ],
        scratch_types=[
            pltpu.VMEM((B,), jnp.int32),
            pltpu.VMEM((B,), jnp.int32),
            pltpu.VMEM((ent_n,), jnp.float32),
            pltpu.VMEM((rel_n,), jnp.float32),
            pltpu.VMEM((_DPW, B), jnp.float32),
            pltpu.VMEM((_BPW,), jnp.float32),
            pltpu.SemaphoreType.DMA,
            pltpu.SemaphoreType.DMA,
            pltpu.SemaphoreType.DMA,
        ],
        compiler_params=pltpu.CompilerParams(use_tc_tiling_on_sc=True,
                                             needs_layout_passes=False),
    )
    def sc_gather(src_hbm, rel_hbm, entt_hbm, relt_hbm, hb_hbm,
                  out_hrt, out_hb,
                  src_v, rel_v, row_v, rrow_v, h_v, hb_v,
                  sem_e, sem_r, sem_b):
        wid = lax.axis_index("s") * _NC + lax.axis_index("c")
        base_d = wid * _DPW
        pltpu.sync_copy(src_hbm, src_v)
        pltpu.sync_copy(rel_hbm, rel_v)
        cp_b = pltpu.async_copy(
            hb_hbm.at[src_v.at[pl.ds(wid * _BPW, _BPW)]], hb_v, sem_b)

        for di in range(_DPW):
            d = base_d + di
            cp_e = pltpu.async_copy(entt_hbm.at[d], row_v, sem_e)
            cp_r = pltpu.async_copy(relt_hbm.at[d], rrow_v, sem_r)
            cp_e.wait()
            cp_r.wait()

            def gather(c, _):
                sl = pl.ds(c * 16, 16)
                h = plsc.load_gather(row_v, [src_v[sl]])
                r = plsc.load_gather(rrow_v, [rel_v[sl]])
                h_v[di, sl] = h * r
                return ()

            lax.fori_loop(0, B // 16, gather, ())

        pltpu.sync_copy(h_v, out_hrt.at[pl.ds(base_d, _DPW)])
        cp_b.wait()
        pltpu.sync_copy(hb_v, out_hb.at[pl.ds(wid * _BPW, _BPW)])

    return sc_gather


def _tc_score(entt_ref, hrt_ref, hb_ref, tail_ref, out_ref):
    acc = lax.dot_general(entt_ref[...], hrt_ref[...], (((0,), (0,)), ((), ())),
                          preferred_element_type=jnp.float32)
    ones_row = jnp.full((1, B), 1.0, dtype=jnp.float32)
    tcol = lax.dot_general(tail_ref[...], ones_row, (((0,), (0,)), ((), ())),
                           preferred_element_type=jnp.float32)
    out_ref[...] = acc + tcol + hb_ref[...]


def kernel(src, rel, ent_embed, rel_embed, head_bias, tail_bias):
    n = ent_embed.shape[0]
    rn = rel_embed.shape[0]
    src_f = src.reshape(B).astype(jnp.int32)
    rel_f = rel.reshape(B).astype(jnp.int32)
    entt = ent_embed.T
    relt = rel_embed.T

    sc_gather = _make_sc_gather(n, rn)
    hrt, hb = sc_gather(src_f, rel_f, entt, relt, head_bias.reshape(n))

    nb = pl.cdiv(n, N_TILE)
    out_t = pl.pallas_call(
        _tc_score,
        grid=(nb,),
        in_specs=[
            pl.BlockSpec((DIM, N_TILE), lambda j: (0, j)),
            pl.BlockSpec((DIM, B), lambda j: (0, 0)),
            pl.BlockSpec((1, B), lambda j: (0, 0)),
            pl.BlockSpec((1, N_TILE), lambda j: (0, j)),
        ],
        out_specs=pl.BlockSpec((N_TILE, B), lambda j: (j, 0)),
        out_shape=jax.ShapeDtypeStruct((n, B), jnp.float32),
        compiler_params=pltpu.CompilerParams(
            dimension_semantics=("parallel",)),
    )(entt, hrt, hb.reshape(1, B), tail_bias.reshape(1, n))
    return out_t.T
